# baseline (speedup 1.0000x reference)
import functools

import jax
import jax.numpy as jnp
from jax import lax
from jax.experimental import pallas as pl
from jax.experimental.pallas import tpu as pltpu
from jax.experimental.pallas import tpu_sc as plsc

MAX_H = 128
P = 2048
D = 512
B = 8
TC_H = 96
SC_H = MAX_H - TC_H
ZROWS = 4

NC = 2
NS = 16
NW = NC * NS
PC = 64
NZCHUNKS = P // PC
NZ = SC_H * NZCHUNKS
PER_W = NZ // NW
LANES = 16
VECS_PER_ROW = D // LANES


def _tc_kernel(state_hbm, out_hbm, zeros_vmem, state_vmem, mean_vmem,
               zsem, ssem, msem):
    zeros_vmem[...] = jnp.zeros_like(zeros_vmem)
    copies = []
    for s in range(1, TC_H, ZROWS):
        r = min(ZROWS, TC_H - s)
        c = pltpu.make_async_copy(
            zeros_vmem.at[pl.ds(0, r)], out_hbm.at[pl.ds(s, r)], zsem)
        c.start()
        copies.append(c)
    in_copy = pltpu.make_async_copy(state_hbm, state_vmem, ssem)
    in_copy.start()
    in_copy.wait()
    mean_vmem[...] = jnp.mean(state_vmem[...], axis=0, keepdims=True)
    m_copy = pltpu.make_async_copy(mean_vmem, out_hbm.at[pl.ds(0, 1)], msem)
    m_copy.start()
    for c in copies:
        c.wait()
    m_copy.wait()


def _sc_kernel(out_hbm, zeros_v, zsem):
    wid = lax.axis_index("s") * NC + lax.axis_index("c")

    def zfill(t, carry):
        r = t // VECS_PER_ROW
        i = t % VECS_PER_ROW
        zeros_v[r, pl.ds(i * LANES, LANES)] = jnp.zeros((LANES,), jnp.float32)
        return carry
    lax.fori_loop(0, PC * VECS_PER_ROW, zfill, None)

    def zfire(j, carry):
        idx = wid * PER_W + j
        h = idx // NZCHUNKS
        pc = idx % NZCHUNKS
        pltpu.async_copy(zeros_v, out_hbm.at[h, pl.ds(pc * PC, PC)], zsem)
        return carry
    lax.fori_loop(0, PER_W, zfire, None)

    def zdrain(j, carry):
        pltpu.make_async_copy(
            zeros_v, out_hbm.at[0, pl.ds(0, PC)], zsem).wait()
        return carry
    lax.fori_loop(0, PER_W, zdrain, None)


def kernel(state):
    if state.ndim == 2:
        state = state[None, :, :]
    top = pl.pallas_call(
        _tc_kernel,
        in_specs=[pl.BlockSpec(memory_space=pltpu.MemorySpace.HBM)],
        out_specs=pl.BlockSpec(memory_space=pltpu.MemorySpace.HBM),
        out_shape=jax.ShapeDtypeStruct((TC_H, P, D), state.dtype),
        scratch_shapes=[
            pltpu.VMEM((ZROWS, P, D), state.dtype),
            pltpu.VMEM((B, P, D), state.dtype),
            pltpu.VMEM((1, P, D), state.dtype),
            pltpu.SemaphoreType.DMA,
            pltpu.SemaphoreType.DMA,
            pltpu.SemaphoreType.DMA,
        ],
    )(state)
    mesh = plsc.VectorSubcoreMesh(core_axis_name="c", subcore_axis_name="s")
    bottom = pl.kernel(
        _sc_kernel,
        mesh=mesh,
        out_type=jax.ShapeDtypeStruct((SC_H, P, D), jnp.float32),
        scratch_types=[
            pltpu.VMEM((PC, D), jnp.float32),
            pltpu.SemaphoreType.DMA,
        ],
    )()
    buf = jnp.concatenate([top, bottom], axis=0)
    return buf, jnp.asarray(1, dtype=jnp.int32)


# speedup vs baseline: 3.0689x; 3.0689x over previous
import jax
import jax.numpy as jnp
from jax.experimental import pallas as pl
from jax.experimental.pallas import tpu as pltpu

MAX_H = 128
ZROWS = 4


def _update_kernel(state_hbm, out_hbm, zeros_vmem, state_vmem, mean_vmem,
                   zsem, ssem, msem):
    zeros_vmem[...] = jnp.zeros_like(zeros_vmem)
    copies = []
    for s in range(1, MAX_H, ZROWS):
        r = min(ZROWS, MAX_H - s)
        c = pltpu.make_async_copy(
            zeros_vmem.at[pl.ds(0, r)], out_hbm.at[pl.ds(s, r)], zsem)
        c.start()
        copies.append(c)
    in_copy = pltpu.make_async_copy(state_hbm, state_vmem, ssem)
    in_copy.start()
    in_copy.wait()
    mean_vmem[...] = jnp.mean(state_vmem[...], axis=0, keepdims=True)
    m_copy = pltpu.make_async_copy(mean_vmem, out_hbm.at[pl.ds(0, 1)], msem)
    m_copy.start()
    for c in copies:
        c.wait()
    m_copy.wait()


def kernel(state):
    if state.ndim == 2:
        state = state[None, :, :]
    B, P, D = state.shape
    buf = pl.pallas_call(
        _update_kernel,
        in_specs=[pl.BlockSpec(memory_space=pltpu.MemorySpace.HBM)],
        out_specs=pl.BlockSpec(memory_space=pltpu.MemorySpace.HBM),
        out_shape=jax.ShapeDtypeStruct((MAX_H, P, D), state.dtype),
        scratch_shapes=[
            pltpu.VMEM((ZROWS, P, D), state.dtype),
            pltpu.VMEM((B, P, D), state.dtype),
            pltpu.VMEM((1, P, D), state.dtype),
            pltpu.SemaphoreType.DMA,
            pltpu.SemaphoreType.DMA,
            pltpu.SemaphoreType.DMA,
        ],
    )(state)
    return buf, jnp.asarray(1, dtype=jnp.int32)


# speedup vs baseline: 3.0842x; 1.0050x over previous
import jax
import jax.numpy as jnp
from jax.experimental import pallas as pl
from jax.experimental.pallas import tpu as pltpu

MAX_H = 128
ZROWS = 2


def _update_kernel(state_hbm, out_hbm, zeros_vmem, state_vmem, mean_vmem,
                   zsem, ssem, msem):
    zeros_vmem[...] = jnp.zeros_like(zeros_vmem)
    copies = []
    for s in range(1, MAX_H, ZROWS):
        r = min(ZROWS, MAX_H - s)
        c = pltpu.make_async_copy(
            zeros_vmem.at[pl.ds(0, r)], out_hbm.at[pl.ds(s, r)], zsem)
        c.start()
        copies.append(c)
    in_copy = pltpu.make_async_copy(state_hbm, state_vmem, ssem)
    in_copy.start()
    in_copy.wait()
    mean_vmem[...] = jnp.mean(state_vmem[...], axis=0, keepdims=True)
    m_copy = pltpu.make_async_copy(mean_vmem, out_hbm.at[pl.ds(0, 1)], msem)
    m_copy.start()
    for c in copies:
        c.wait()
    m_copy.wait()


def kernel(state):
    if state.ndim == 2:
        state = state[None, :, :]
    B, P, D = state.shape
    buf = pl.pallas_call(
        _update_kernel,
        in_specs=[pl.BlockSpec(memory_space=pltpu.MemorySpace.HBM)],
        out_specs=pl.BlockSpec(memory_space=pltpu.MemorySpace.HBM),
        out_shape=jax.ShapeDtypeStruct((MAX_H, P, D), state.dtype),
        scratch_shapes=[
            pltpu.VMEM((ZROWS, P, D), state.dtype),
            pltpu.VMEM((B, P, D), state.dtype),
            pltpu.VMEM((1, P, D), state.dtype),
            pltpu.SemaphoreType.DMA,
            pltpu.SemaphoreType.DMA,
            pltpu.SemaphoreType.DMA,
        ],
    )(state)
    return buf, jnp.asarray(1, dtype=jnp.int32)
